# baseline (speedup 1.0000x reference)
import functools

import jax
import jax.numpy as jnp
import numpy as np
from jax import lax
from jax.experimental import pallas as pl
from jax.experimental.pallas import tpu as pltpu

N_NODES = 10000
N_EDGES = 160000
D = 256
D_OUT = 16
E_EXP = 64
TOP_K = 2
D_FF = 512
CAP = 1024

BLK = 400
NTILES = N_NODES // BLK
FBLK = 256
MAXTILES = (N_NODES * TOP_K + E_EXP * (FBLK - 1)) // FBLK + 1
MAXROWS = MAXTILES * FBLK
DUMMY = MAXROWS
NA_PAD = 20480
NZ_PAD = NA_PAD // 2


def _gelu(x):
    return 0.5 * x * (1.0 + lax.erf(x / np.sqrt(2.0).astype(np.float32)))


def _stage_a_body(h_ref, cnt_ref, stats_ref, g_ref, b_ref, wg_ref, wr_ref,
                  hws_ref, hres_ref):
    hn = (h_ref[...] - stats_ref[0:1, :]) * stats_ref[1:2, :] * g_ref[...] + b_ref[...]
    dinv = lax.rsqrt(cnt_ref[...] + 1.0)
    hws_ref[...] = jnp.dot(hn, wg_ref[...], preferred_element_type=jnp.float32) * dinv
    hres_ref[...] = jnp.dot(hn, wr_ref[...], preferred_element_type=jnp.float32)


def _stage_a(h, cnt, stats, bn_g, bn_b, Wg, Wres):
    return pl.pallas_call(
        _stage_a_body,
        grid=(NTILES,),
        in_specs=[
            pl.BlockSpec((BLK, D), lambda i: (i, 0)),
            pl.BlockSpec((BLK, 1), lambda i: (i, 0)),
            pl.BlockSpec((8, D), lambda i: (0, 0)),
            pl.BlockSpec((1, D), lambda i: (0, 0)),
            pl.BlockSpec((1, D), lambda i: (0, 0)),
            pl.BlockSpec((D, D), lambda i: (0, 0)),
            pl.BlockSpec((D, D), lambda i: (0, 0)),
        ],
        out_specs=[
            pl.BlockSpec((BLK, D), lambda i: (i, 0)),
            pl.BlockSpec((BLK, D), lambda i: (i, 0)),
        ],
        out_shape=[
            jax.ShapeDtypeStruct((N_NODES, D), jnp.float32),
            jax.ShapeDtypeStruct((N_NODES, D), jnp.float32),
        ],
    )(h, cnt, stats, bn_g, bn_b, Wg, Wres)


def _stage_c_body(agg_ref, hws_ref, hres_ref, cnt_ref, bg_ref, bres_ref,
                  gw_ref, gb_ref, tril_ref,
                  z_ref, ti_ref, pos_ref, sc_ref, cto_ref, carry_ref):
    i = pl.program_id(0)

    @pl.when(i == 0)
    def _():
        carry_ref[...] = jnp.zeros_like(carry_ref)

    dinv = lax.rsqrt(cnt_ref[...] + 1.0)
    z = ((agg_ref[...] * dinv + hws_ref[...] * dinv) + bg_ref[...]
         + hres_ref[...] + bres_ref[...])
    z = jnp.maximum(z, 0.0)
    z_ref[...] = z

    logits = jnp.dot(z, gw_ref[...], preferred_element_type=jnp.float32) + gb_ref[...]
    iota = lax.broadcasted_iota(jnp.int32, (BLK, E_EXP), 1)
    m1 = jnp.max(logits, axis=1, keepdims=True)
    i1 = jnp.min(jnp.where(logits == m1, iota, E_EXP), axis=1, keepdims=True)
    l2 = jnp.where(iota == i1, -jnp.inf, logits)
    m2 = jnp.max(l2, axis=1, keepdims=True)
    i2 = jnp.min(jnp.where(l2 == m2, iota, E_EXP), axis=1, keepdims=True)
    t = jnp.exp(m2 - m1)
    g1 = 1.0 / (1.0 + t)
    g2 = t * g1

    oh1 = (iota == i1).astype(jnp.float32)
    oh2 = (iota == i2).astype(jnp.float32)
    oh12 = oh1 + oh2
    pre = jnp.dot(tril_ref[...], oh12, preferred_element_type=jnp.float32,
                  precision=lax.Precision.HIGHEST)
    posmat = pre + carry_ref[0:1, :]
    pos1 = jnp.sum(posmat * oh1, axis=1, keepdims=True)
    pos2 = jnp.sum(posmat * oh2, axis=1, keepdims=True)
    carry_ref[0:1, :] += jnp.sum(oh12, axis=0, keepdims=True)

    ti_ref[...] = jnp.concatenate([i1, i2], axis=1)
    pos_ref[...] = jnp.concatenate([pos1, pos2], axis=1).astype(jnp.int32)
    v1 = (pos1 < CAP).astype(jnp.float32)
    v2 = (pos2 < CAP).astype(jnp.float32)
    sc_ref[...] = jnp.concatenate([g1 * v1, g2 * v2], axis=1)
    cto_ref[...] = carry_ref[...]


def _stage_c(agg, hws, hres, cnt, bg, bres, gw, gb, tril):
    return pl.pallas_call(
        _stage_c_body,
        grid=(NTILES,),
        in_specs=[
            pl.BlockSpec((BLK, D), lambda i: (i, 0)),
            pl.BlockSpec((BLK, D), lambda i: (i, 0)),
            pl.BlockSpec((BLK, D), lambda i: (i, 0)),
            pl.BlockSpec((BLK, 1), lambda i: (i, 0)),
            pl.BlockSpec((1, D), lambda i: (0, 0)),
            pl.BlockSpec((1, D), lambda i: (0, 0)),
            pl.BlockSpec((D, E_EXP), lambda i: (0, 0)),
            pl.BlockSpec((1, E_EXP), lambda i: (0, 0)),
            pl.BlockSpec((BLK, BLK), lambda i: (0, 0)),
        ],
        out_specs=[
            pl.BlockSpec((BLK, D), lambda i: (i, 0)),
            pl.BlockSpec((BLK, 2), lambda i: (i, 0)),
            pl.BlockSpec((BLK, 2), lambda i: (i, 0)),
            pl.BlockSpec((BLK, 2), lambda i: (i, 0)),
            pl.BlockSpec((8, E_EXP), lambda i: (0, 0)),
        ],
        out_shape=[
            jax.ShapeDtypeStruct((NZ_PAD, D), jnp.float32),
            jax.ShapeDtypeStruct((N_NODES, 2), jnp.int32),
            jax.ShapeDtypeStruct((N_NODES, 2), jnp.int32),
            jax.ShapeDtypeStruct((N_NODES, 2), jnp.float32),
            jax.ShapeDtypeStruct((8, E_EXP), jnp.float32),
        ],
        scratch_shapes=[pltpu.VMEM((8, E_EXP), jnp.float32)],
    )(agg, hws, hres, cnt, bg, bres, gw, gb, tril)


def _stage_e_body(eot_ref, nt_ref, x_ref, w1_ref, b1_ref, w2_ref, b2_ref, o_ref):
    @pl.when(pl.program_id(0) < nt_ref[0])
    def _():
        mid = _gelu(jnp.dot(x_ref[...], w1_ref[0],
                            preferred_element_type=jnp.float32) + b1_ref[0])
        o_ref[...] = jnp.dot(mid, w2_ref[0],
                             preferred_element_type=jnp.float32) + b2_ref[0]


def _stage_e(eot, nt, xbuf, W1, b1, W2, b2):
    grid_spec = pltpu.PrefetchScalarGridSpec(
        num_scalar_prefetch=2,
        grid=(MAXTILES,),
        in_specs=[
            pl.BlockSpec((FBLK, D), lambda t, eot, nt: (t, 0)),
            pl.BlockSpec((1, D, D_FF), lambda t, eot, nt: (eot[t], 0, 0)),
            pl.BlockSpec((1, 1, D_FF), lambda t, eot, nt: (eot[t], 0, 0)),
            pl.BlockSpec((1, D_FF, D), lambda t, eot, nt: (eot[t], 0, 0)),
            pl.BlockSpec((1, 1, D), lambda t, eot, nt: (eot[t], 0, 0)),
        ],
        out_specs=pl.BlockSpec((FBLK, D), lambda t, eot, nt: (t, 0)),
    )
    return pl.pallas_call(
        _stage_e_body,
        grid_spec=grid_spec,
        out_shape=jax.ShapeDtypeStruct((MAXROWS + 8, D), jnp.float32),
    )(eot, nt, xbuf, W1, b1, W2, b2)


def _stage_g_body(z_ref, o_ref):
    @pl.when(pl.program_id(0) == 0)
    def _():
        o_ref[...] = jnp.zeros_like(o_ref)
    z = z_ref[...]
    o_ref[0:1, :] += jnp.sum(z, axis=0, keepdims=True)
    o_ref[1:2, :] += jnp.sum(z * z, axis=0, keepdims=True)


def _stage_g(zc):
    return pl.pallas_call(
        _stage_g_body,
        grid=(NTILES,),
        in_specs=[pl.BlockSpec((BLK, D), lambda i: (i, 0))],
        out_specs=pl.BlockSpec((8, D), lambda i: (0, 0)),
        out_shape=jax.ShapeDtypeStruct((8, D), jnp.float32),
    )(zc)


def _final_body(z_ref, stats_ref, g_ref, b_ref, wf_ref, bf_ref, o_ref):
    hn = (z_ref[...] - stats_ref[0:1, :]) * stats_ref[1:2, :] * g_ref[...] + b_ref[...]
    o_ref[...] = jnp.dot(hn, wf_ref[...], preferred_element_type=jnp.float32) + bf_ref[...]


def _final(zc, stats, bn_g, bn_b, Wf, bf):
    return pl.pallas_call(
        _final_body,
        grid=(NTILES,),
        in_specs=[
            pl.BlockSpec((BLK, D), lambda i: (i, 0)),
            pl.BlockSpec((8, D), lambda i: (0, 0)),
            pl.BlockSpec((1, D), lambda i: (0, 0)),
            pl.BlockSpec((1, D), lambda i: (0, 0)),
            pl.BlockSpec((D, D_OUT), lambda i: (0, 0)),
            pl.BlockSpec((1, D_OUT), lambda i: (0, 0)),
        ],
        out_specs=pl.BlockSpec((BLK, D_OUT), lambda i: (i, 0)),
        out_shape=jax.ShapeDtypeStruct((N_NODES, D_OUT), jnp.float32),
    )(zc, stats, bn_g, bn_b, Wf, bf)


def _sc_deg(dst):
    cnt = jnp.bincount(dst, length=N_NODES).astype(jnp.float32)
    return jnp.pad(cnt, (0, 16))


def _sc_spmm(hws, src, dst):
    agg = jax.ops.segment_sum(hws[src], dst, num_segments=N_NODES)
    return jnp.pad(agg, ((0, 16), (0, 0)))


def _sc_dispatch(z, ef, posf, pstart):
    slotp = jnp.where(posf < CAP, pstart[ef] + posf, DUMMY)
    tok = jnp.arange(NA_PAD, dtype=jnp.int32) // 2
    xbuf = jnp.zeros((MAXROWS + 8, D), jnp.float32).at[slotp].set(z[tok])
    return xbuf, slotp


def _sc_combine(ybuf, z, slotp, scf):
    rows = ybuf[slotp]
    sc = scf[:, None]
    contrib = jnp.where(sc > 0, rows * sc, 0.0)
    core = contrib.reshape(NZ_PAD, 2, D).sum(axis=1)
    return z + core


_TRIL = None


def _get_tril():
    global _TRIL
    if _TRIL is None:
        _TRIL = jnp.asarray(np.tril(np.ones((BLK, BLK), np.float32), k=-1))
    return _TRIL


def kernel(x, params, edge_index):
    src = edge_index[0]
    dst = edge_index[1]
    tril = _get_tril()

    cnt = _sc_deg(dst)
    cnt_col = cnt[:, None]

    ident = jnp.concatenate([
        jnp.zeros((1, D), jnp.float32),
        jnp.ones((1, D), jnp.float32),
        jnp.zeros((6, D), jnp.float32),
    ], axis=0)
    one_row = jnp.ones((1, D), jnp.float32)
    zero_row = jnp.zeros((1, D), jnp.float32)

    h = x
    stats, g_row, b_row = ident, one_row, zero_row
    for p in params['layers']:
        hws, hres = _stage_a(h, cnt_col, stats, g_row, b_row, p['Wg'], p['Wres'])
        aggraw = _sc_spmm(hws, src, dst)
        z, ti, pos, sc, cto = _stage_c(
            aggraw, hws, hres, cnt_col, p['bg'][None, :], p['bres'][None, :],
            p['gate_W'], p['gate_b'][None, :], tril)

        counts = cto[0].astype(jnp.int32)
        clamped = jnp.minimum(counts, CAP)
        padded = ((clamped + FBLK - 1) // FBLK) * FBLK
        pstart = jnp.cumsum(padded) - padded
        nt = jnp.sum(padded) // FBLK
        tiles_per = padded // FBLK
        eot = jnp.repeat(jnp.arange(E_EXP, dtype=jnp.int32), tiles_per,
                         total_repeat_length=MAXTILES)

        ef = jnp.pad(ti.reshape(-1), (0, NA_PAD - N_NODES * 2))
        posf = jnp.pad(pos.reshape(-1), (0, NA_PAD - N_NODES * 2),
                       constant_values=CAP)
        scf = jnp.pad(sc.reshape(-1), (0, NA_PAD - N_NODES * 2))

        xbuf, slotp = _sc_dispatch(z, ef, posf, pstart.astype(jnp.int32))
        ybuf = _stage_e(eot, nt.astype(jnp.int32)[None], xbuf,
                        p['W1'], p['b1'][:, None, :], p['W2'], p['b2'][:, None, :])
        zc = _sc_combine(ybuf, z, slotp, scf)

        sums = _stage_g(zc)
        mean = sums[0:1, :] / N_NODES
        var = sums[1:2, :] / N_NODES - mean * mean
        rstd = lax.rsqrt(var + 1e-5)
        stats = jnp.concatenate([mean, rstd, jnp.zeros((6, D), jnp.float32)], axis=0)
        g_row, b_row = p['bn_g'][None, :], p['bn_b'][None, :]
        h = zc

    return _final(h, stats, g_row, b_row,
                  params['final']['Wf'], params['final']['bf'][None, :])


# speedup vs baseline: 1.1075x; 1.1075x over previous
import functools

import jax
import jax.numpy as jnp
import numpy as np
from jax import lax
from jax.experimental import pallas as pl
from jax.experimental.pallas import tpu as pltpu
from jax.experimental.pallas import tpu_sc as plsc

N_NODES = 10000
N_EDGES = 160000
D = 256
D_OUT = 16
E_EXP = 64
TOP_K = 2
D_FF = 512
CAP = 1024

BLK = 400
NTILES = N_NODES // BLK
FBLK = 256
MAXTILES = (N_NODES * TOP_K + E_EXP * (FBLK - 1)) // FBLK + 1
MAXROWS = MAXTILES * FBLK
DUMMY = MAXROWS
NA_PAD = 20480
NZ_PAD = NA_PAD // 2


def _gelu(x):
    return 0.5 * x * (1.0 + lax.erf(x / np.sqrt(2.0).astype(np.float32)))


def _stage_a_body(h_ref, cnt_ref, stats_ref, g_ref, b_ref, wg_ref, wr_ref,
                  hws_ref, hres_ref):
    hn = (h_ref[...] - stats_ref[0:1, :]) * stats_ref[1:2, :] * g_ref[...] + b_ref[...]
    dinv = lax.rsqrt(cnt_ref[...] + 1.0)
    hws_ref[...] = jnp.dot(hn, wg_ref[...], preferred_element_type=jnp.float32) * dinv
    hres_ref[...] = jnp.dot(hn, wr_ref[...], preferred_element_type=jnp.float32)


def _stage_a(h, cnt, stats, bn_g, bn_b, Wg, Wres):
    return pl.pallas_call(
        _stage_a_body,
        grid=(NTILES,),
        in_specs=[
            pl.BlockSpec((BLK, D), lambda i: (i, 0)),
            pl.BlockSpec((BLK, 1), lambda i: (i, 0)),
            pl.BlockSpec((8, D), lambda i: (0, 0)),
            pl.BlockSpec((1, D), lambda i: (0, 0)),
            pl.BlockSpec((1, D), lambda i: (0, 0)),
            pl.BlockSpec((D, D), lambda i: (0, 0)),
            pl.BlockSpec((D, D), lambda i: (0, 0)),
        ],
        out_specs=[
            pl.BlockSpec((BLK, D), lambda i: (i, 0)),
            pl.BlockSpec((BLK, D), lambda i: (i, 0)),
        ],
        out_shape=[
            jax.ShapeDtypeStruct((N_NODES, D), jnp.float32),
            jax.ShapeDtypeStruct((N_NODES, D), jnp.float32),
        ],
    )(h, cnt, stats, bn_g, bn_b, Wg, Wres)


def _stage_c_body(agg_ref, hws_ref, hres_ref, cnt_ref, bg_ref, bres_ref,
                  gw_ref, gb_ref, tril_ref,
                  z_ref, ti_ref, pos_ref, sc_ref, cto_ref, carry_ref):
    i = pl.program_id(0)

    @pl.when(i == 0)
    def _():
        carry_ref[...] = jnp.zeros_like(carry_ref)

    dinv = lax.rsqrt(cnt_ref[...] + 1.0)
    z = ((agg_ref[...] * dinv + hws_ref[...] * dinv) + bg_ref[...]
         + hres_ref[...] + bres_ref[...])
    z = jnp.maximum(z, 0.0)
    z_ref[...] = z

    logits = jnp.dot(z, gw_ref[...], preferred_element_type=jnp.float32) + gb_ref[...]
    iota = lax.broadcasted_iota(jnp.int32, (BLK, E_EXP), 1)
    m1 = jnp.max(logits, axis=1, keepdims=True)
    i1 = jnp.min(jnp.where(logits == m1, iota, E_EXP), axis=1, keepdims=True)
    l2 = jnp.where(iota == i1, -jnp.inf, logits)
    m2 = jnp.max(l2, axis=1, keepdims=True)
    i2 = jnp.min(jnp.where(l2 == m2, iota, E_EXP), axis=1, keepdims=True)
    t = jnp.exp(m2 - m1)
    g1 = 1.0 / (1.0 + t)
    g2 = t * g1

    oh1 = (iota == i1).astype(jnp.float32)
    oh2 = (iota == i2).astype(jnp.float32)
    oh12 = oh1 + oh2
    pre = jnp.dot(tril_ref[...], oh12, preferred_element_type=jnp.float32,
                  precision=lax.Precision.HIGHEST)
    posmat = pre + carry_ref[0:1, :]
    pos1 = jnp.sum(posmat * oh1, axis=1, keepdims=True)
    pos2 = jnp.sum(posmat * oh2, axis=1, keepdims=True)
    carry_ref[0:1, :] += jnp.sum(oh12, axis=0, keepdims=True)

    ti_ref[...] = jnp.concatenate([i1, i2], axis=1)
    pos_ref[...] = jnp.concatenate([pos1, pos2], axis=1).astype(jnp.int32)
    v1 = (pos1 < CAP).astype(jnp.float32)
    v2 = (pos2 < CAP).astype(jnp.float32)
    sc_ref[...] = jnp.concatenate([g1 * v1, g2 * v2], axis=1)
    cto_ref[...] = carry_ref[...]


def _stage_c(agg, hws, hres, cnt, bg, bres, gw, gb, tril):
    return pl.pallas_call(
        _stage_c_body,
        grid=(NTILES,),
        in_specs=[
            pl.BlockSpec((BLK, D), lambda i: (i, 0)),
            pl.BlockSpec((BLK, D), lambda i: (i, 0)),
            pl.BlockSpec((BLK, D), lambda i: (i, 0)),
            pl.BlockSpec((BLK, 1), lambda i: (i, 0)),
            pl.BlockSpec((1, D), lambda i: (0, 0)),
            pl.BlockSpec((1, D), lambda i: (0, 0)),
            pl.BlockSpec((D, E_EXP), lambda i: (0, 0)),
            pl.BlockSpec((1, E_EXP), lambda i: (0, 0)),
            pl.BlockSpec((BLK, BLK), lambda i: (0, 0)),
        ],
        out_specs=[
            pl.BlockSpec((BLK, D), lambda i: (i, 0)),
            pl.BlockSpec((BLK, 2), lambda i: (i, 0)),
            pl.BlockSpec((BLK, 2), lambda i: (i, 0)),
            pl.BlockSpec((BLK, 2), lambda i: (i, 0)),
            pl.BlockSpec((8, E_EXP), lambda i: (0, 0)),
        ],
        out_shape=[
            jax.ShapeDtypeStruct((NZ_PAD, D), jnp.float32),
            jax.ShapeDtypeStruct((N_NODES, 2), jnp.int32),
            jax.ShapeDtypeStruct((N_NODES, 2), jnp.int32),
            jax.ShapeDtypeStruct((N_NODES, 2), jnp.float32),
            jax.ShapeDtypeStruct((8, E_EXP), jnp.float32),
        ],
        scratch_shapes=[pltpu.VMEM((8, E_EXP), jnp.float32)],
    )(agg, hws, hres, cnt, bg, bres, gw, gb, tril)


def _stage_e_body(eot_ref, nt_ref, x_ref, w1_ref, b1_ref, w2_ref, b2_ref, o_ref):
    t = pl.program_id(0)

    @pl.when(t < nt_ref[0])
    def _():
        mid = _gelu(jnp.dot(x_ref[...], w1_ref[0],
                            preferred_element_type=jnp.float32) + b1_ref[0])
        o_ref[...] = jnp.dot(mid, w2_ref[0],
                             preferred_element_type=jnp.float32) + b2_ref[0]

    @pl.when(t >= nt_ref[0])
    def _():
        o_ref[...] = jnp.zeros_like(o_ref)


def _stage_e(eot, nt, xbuf, W1, b1, W2, b2):
    grid_spec = pltpu.PrefetchScalarGridSpec(
        num_scalar_prefetch=2,
        grid=(MAXTILES + 1,),
        in_specs=[
            pl.BlockSpec((FBLK, D), lambda t, eot, nt: (t, 0)),
            pl.BlockSpec((1, D, D_FF), lambda t, eot, nt: (eot[t], 0, 0)),
            pl.BlockSpec((1, 1, D_FF), lambda t, eot, nt: (eot[t], 0, 0)),
            pl.BlockSpec((1, D_FF, D), lambda t, eot, nt: (eot[t], 0, 0)),
            pl.BlockSpec((1, 1, D), lambda t, eot, nt: (eot[t], 0, 0)),
        ],
        out_specs=pl.BlockSpec((FBLK, D), lambda t, eot, nt: (t, 0)),
    )
    return pl.pallas_call(
        _stage_e_body,
        grid_spec=grid_spec,
        out_shape=jax.ShapeDtypeStruct((MAXROWS + 8, D), jnp.float32),
    )(eot, nt, xbuf, W1, b1, W2, b2)


def _stage_g_body(z_ref, o_ref):
    @pl.when(pl.program_id(0) == 0)
    def _():
        o_ref[...] = jnp.zeros_like(o_ref)
    z = z_ref[...]
    o_ref[0:1, :] += jnp.sum(z, axis=0, keepdims=True)
    o_ref[1:2, :] += jnp.sum(z * z, axis=0, keepdims=True)


def _stage_g(zc):
    return pl.pallas_call(
        _stage_g_body,
        grid=(NTILES,),
        in_specs=[pl.BlockSpec((BLK, D), lambda i: (i, 0))],
        out_specs=pl.BlockSpec((8, D), lambda i: (0, 0)),
        out_shape=jax.ShapeDtypeStruct((8, D), jnp.float32),
    )(zc)


def _final_body(z_ref, stats_ref, g_ref, b_ref, wf_ref, bf_ref, o_ref):
    hn = (z_ref[...] - stats_ref[0:1, :]) * stats_ref[1:2, :] * g_ref[...] + b_ref[...]
    o_ref[...] = jnp.dot(hn, wf_ref[...], preferred_element_type=jnp.float32) + bf_ref[...]


def _final(zc, stats, bn_g, bn_b, Wf, bf):
    return pl.pallas_call(
        _final_body,
        grid=(NTILES,),
        in_specs=[
            pl.BlockSpec((BLK, D), lambda i: (i, 0)),
            pl.BlockSpec((8, D), lambda i: (0, 0)),
            pl.BlockSpec((1, D), lambda i: (0, 0)),
            pl.BlockSpec((1, D), lambda i: (0, 0)),
            pl.BlockSpec((D, D_OUT), lambda i: (0, 0)),
            pl.BlockSpec((1, D_OUT), lambda i: (0, 0)),
        ],
        out_specs=pl.BlockSpec((BLK, D_OUT), lambda i: (i, 0)),
        out_shape=jax.ShapeDtypeStruct((N_NODES, D_OUT), jnp.float32),
    )(zc, stats, bn_g, bn_b, Wf, bf)


def _sc_deg(dst):
    cnt = jnp.bincount(dst, length=N_NODES).astype(jnp.float32)
    return jnp.pad(cnt, (0, 16))


def _sc_spmm(hws, src, dst):
    agg = jax.ops.segment_sum(hws[src], dst, num_segments=N_NODES)
    return jnp.pad(agg, ((0, 16), (0, 0)))


_MESH = plsc.VectorSubcoreMesh(core_axis_name="c", subcore_axis_name="s")
NSC = 2
NTEC = 16
APT = NA_PAD // (NSC * NTEC)
ACH = 10
TPT = NZ_PAD // (NSC * NTEC)


@functools.partial(
    pl.kernel,
    out_type=jax.ShapeDtypeStruct((MAXROWS + 8, D), jnp.float32),
    mesh=_MESH,
    scratch_types=[
        pltpu.VMEM((ACH, 64), jnp.int32),
        pltpu.VMEM((ACH, 64), jnp.int32),
        pltpu.VMEM((64, D), jnp.float32),
        pltpu.VMEM((64, D), jnp.float32),
        pltpu.SemaphoreType.DMA,
        pltpu.SemaphoreType.DMA,
        pltpu.SemaphoreType.DMA,
        pltpu.SemaphoreType.DMA,
    ],
)
def _sc_dispatch_kernel(z_hbm, tok_hbm, slotp_hbm, xbuf_hbm,
                        tok2d, sp2d, rows0, rows1, gsem0, gsem1, ssem0, ssem1):
    c = lax.axis_index("c")
    s = lax.axis_index("s")
    base = (c * NTEC + s) * APT
    for k in range(ACH):
        pltpu.sync_copy(tok_hbm.at[pl.ds(base + 64 * k, 64)], tok2d.at[k])
        pltpu.sync_copy(slotp_hbm.at[pl.ds(base + 64 * k, 64)], sp2d.at[k])
    rows = (rows0, rows1)
    gsem = (gsem0, gsem1)
    ssem = (ssem0, ssem1)
    pltpu.async_copy(z_hbm.at[tok2d.at[0]], rows0, gsem0)
    pltpu.async_copy(z_hbm.at[tok2d.at[1]], rows1, gsem1)
    for k in range(ACH):
        b = k & 1
        pltpu.make_async_copy(z_hbm.at[tok2d.at[k]], rows[b], gsem[b]).wait()
        pltpu.async_copy(rows[b], xbuf_hbm.at[sp2d.at[k]], ssem[b])
        if k + 2 < ACH:
            pltpu.make_async_copy(rows[b], xbuf_hbm.at[sp2d.at[k]], ssem[b]).wait()
            pltpu.async_copy(z_hbm.at[tok2d.at[k + 2]], rows[b], gsem[b])
    pltpu.make_async_copy(rows[0], xbuf_hbm.at[sp2d.at[ACH - 2]], ssem[0]).wait()
    pltpu.make_async_copy(rows[1], xbuf_hbm.at[sp2d.at[ACH - 1]], ssem[1]).wait()


@functools.partial(
    pl.kernel,
    out_type=jax.ShapeDtypeStruct((NZ_PAD, D), jnp.float32),
    mesh=_MESH,
    scratch_types=[
        pltpu.VMEM((APT,), jnp.int32),
        pltpu.VMEM((APT + 16,), jnp.float32),
        pltpu.VMEM((32, D), jnp.float32),
        pltpu.VMEM((64, D), jnp.float32),
        pltpu.VMEM((64, D), jnp.float32),
        pltpu.VMEM((32, D), jnp.float32),
        pltpu.SemaphoreType.DMA,
        pltpu.SemaphoreType.DMA,
    ],
)
def _sc_combine_kernel(ybuf_hbm, z_hbm, slotp_hbm, scf_hbm, zc_hbm,
                       spl, scl, zrows, ybr0, ybr1, outr, gsem0, gsem1):
    c = lax.axis_index("c")
    s = lax.axis_index("s")
    w = c * NTEC + s
    abase = w * APT
    tbase = w * TPT
    pltpu.sync_copy(slotp_hbm.at[pl.ds(abase, APT)], spl)
    pltpu.sync_copy(scf_hbm.at[pl.ds(abase, APT)], scl.at[pl.ds(0, APT)])
    ybr = (ybr0, ybr1)
    gsem = (gsem0, gsem1)
    pltpu.async_copy(ybuf_hbm.at[spl.at[pl.ds(0, 64)]], ybr0, gsem0)
    pltpu.async_copy(ybuf_hbm.at[spl.at[pl.ds(64, 64)]], ybr1, gsem1)
    for k in range(ACH):
        b = k & 1
        pltpu.make_async_copy(ybuf_hbm.at[spl.at[pl.ds(0, 64)]],
                              ybr[b], gsem[b]).wait()
        pltpu.sync_copy(z_hbm.at[pl.ds(tbase + 32 * k, 32)], zrows)

        def rbody(r, carry):
            pair = scl[pl.ds(64 * k + 2 * r, 16)]
            s0 = jnp.full((16,), pair[0])
            s1 = jnp.full((16,), pair[1])
            for j in range(D // 16):
                sl = pl.ds(j * 16, 16)
                y0 = ybr[b][2 * r, sl]
                y1 = ybr[b][2 * r + 1, sl]
                outr[r, sl] = zrows[r, sl] + y0 * s0 + y1 * s1
            return carry
        lax.fori_loop(0, 32, rbody, 0)
        pltpu.sync_copy(outr, zc_hbm.at[pl.ds(tbase + 32 * k, 32)])
        if k + 2 < ACH:
            pltpu.async_copy(ybuf_hbm.at[spl.at[pl.ds((k + 2) * 64, 64)]],
                             ybr[b], gsem[b])


_TOK = None


def _get_tok():
    global _TOK
    if _TOK is None:
        _TOK = jnp.asarray(np.arange(NA_PAD, dtype=np.int32) // 2)
    return _TOK


def _sc_dispatch(z, ef, posf, pstart):
    slotp = jnp.where(posf < CAP, pstart[ef] + posf, DUMMY)
    xbuf = _sc_dispatch_kernel(z, _get_tok(), slotp)
    return xbuf, slotp


def _sc_combine(ybuf, z, slotp, scf):
    return _sc_combine_kernel(ybuf, z, slotp, scf)


_TRIL = None


def _get_tril():
    global _TRIL
    if _TRIL is None:
        _TRIL = jnp.asarray(np.tril(np.ones((BLK, BLK), np.float32), k=-1))
    return _TRIL


def kernel(x, params, edge_index):
    src = edge_index[0]
    dst = edge_index[1]
    tril = _get_tril()

    cnt = _sc_deg(dst)
    cnt_col = cnt[:, None]

    ident = jnp.concatenate([
        jnp.zeros((1, D), jnp.float32),
        jnp.ones((1, D), jnp.float32),
        jnp.zeros((6, D), jnp.float32),
    ], axis=0)
    one_row = jnp.ones((1, D), jnp.float32)
    zero_row = jnp.zeros((1, D), jnp.float32)

    h = x
    stats, g_row, b_row = ident, one_row, zero_row
    for p in params['layers']:
        hws, hres = _stage_a(h, cnt_col, stats, g_row, b_row, p['Wg'], p['Wres'])
        aggraw = _sc_spmm(hws, src, dst)
        z, ti, pos, sc, cto = _stage_c(
            aggraw, hws, hres, cnt_col, p['bg'][None, :], p['bres'][None, :],
            p['gate_W'], p['gate_b'][None, :], tril)

        counts = cto[0].astype(jnp.int32)
        clamped = jnp.minimum(counts, CAP)
        padded = ((clamped + FBLK - 1) // FBLK) * FBLK
        pstart = jnp.cumsum(padded) - padded
        nt = jnp.sum(padded) // FBLK
        tiles_per = padded // FBLK
        eot = jnp.repeat(jnp.arange(E_EXP, dtype=jnp.int32), tiles_per,
                         total_repeat_length=MAXTILES + 1)

        ef = jnp.pad(ti.reshape(-1), (0, NA_PAD - N_NODES * 2))
        posf = jnp.pad(pos.reshape(-1), (0, NA_PAD - N_NODES * 2),
                       constant_values=CAP)
        scf = jnp.pad(sc.reshape(-1), (0, NA_PAD - N_NODES * 2))

        xbuf, slotp = _sc_dispatch(z, ef, posf, pstart.astype(jnp.int32))
        ybuf = _stage_e(eot, nt.astype(jnp.int32)[None], xbuf,
                        p['W1'], p['b1'][:, None, :], p['W2'], p['b2'][:, None, :])
        zc = _sc_combine(ybuf, z, slotp, scf)

        sums = _stage_g(zc)
        mean = sums[0:1, :] / N_NODES
        var = sums[1:2, :] / N_NODES - mean * mean
        rstd = lax.rsqrt(var + 1e-5)
        stats = jnp.concatenate([mean, rstd, jnp.zeros((6, D), jnp.float32)], axis=0)
        g_row, b_row = p['bn_g'][None, :], p['bn_b'][None, :]
        h = zc

    return _final(h, stats, g_row, b_row,
                  params['final']['Wf'], params['final']['bf'][None, :])
